# Initial kernel scaffold; baseline (speedup 1.0000x reference)
import jax
import jax.numpy as jnp
from jax.experimental import pallas as pl


def kernel(in_tensor, hash_table):
    raise NotImplementedError("write your pallas kernel here")


# speedup vs baseline: 1.2513x; 1.2513x over previous
import functools

import jax
import jax.numpy as jnp
import numpy as np
from jax import lax
from jax.experimental import pallas as pl
from jax.experimental.pallas import tpu as pltpu
from jax.experimental.pallas import tpu_sc as plsc

NUM_LEVELS = 16
MIN_RES = 16
MAX_RES = 4096
LOG2_HASHMAP_SIZE = 19
FEATS = 2
TABLE_SIZE = 2 ** LOG2_HASHMAP_SIZE
N_POINTS = 262144

NC = 2
NS = 16
NW = NC * NS
NPT = N_POINTS // NW
CH = 512
NCHUNK = NPT // CH
NGRP = CH // 16
NROW = 8 * CH // 128

MASK = np.int32(TABLE_SIZE - 1)
P2 = np.int32(np.uint32(2654435761))
P3 = np.int32(805459861)

_growth = np.exp((np.log(MAX_RES) - np.log(MIN_RES)) / (NUM_LEVELS - 1))
SCALES = [np.float32(np.floor(MIN_RES * _growth ** l)) for l in range(NUM_LEVELS)]


def _body(in_hbm, table_hbm, out_hbm, coords, idxbuf, rows, wbuf, outbuf,
          sem0, sem1):
    sems = (sem0, sem1)
    wid = lax.axis_index("s") * NC + lax.axis_index("c")

    iota = jnp.arange(16, dtype=jnp.int32)
    dup = iota >> 1
    fpat = iota & 1
    col0 = jnp.zeros((16,), jnp.int32)
    col1 = jnp.full((16,), 1, jnp.int32)
    col2 = jnp.full((16,), 2, jnp.int32)

    def hash_level(l, par):
        scale = SCALES[l]
        off_l = np.int32(l * TABLE_SIZE)

        def grp(g, _):
            p16 = g * np.int32(16)
            rowidx = p16 + iota
            xv = plsc.load_gather(coords, [rowidx, col0])
            yv = plsc.load_gather(coords, [rowidx, col1])
            zv = plsc.load_gather(coords, [rowidx, col2])
            one = np.int32(1)
            zero = np.int32(0)

            sx = xv * scale
            fx = sx.astype(jnp.int32)
            wx = sx - fx.astype(jnp.float32)
            cx = fx + jnp.where(wx > 0, one, zero)

            sy = yv * scale
            fy = sy.astype(jnp.int32)
            wy = sy - fy.astype(jnp.float32)
            cy = fy + jnp.where(wy > 0, one, zero)

            sz = zv * scale
            fz = sz.astype(jnp.int32)
            wz = sz - fz.astype(jnp.float32)
            cz = fz + jnp.where(wz > 0, one, zero)

            wb = np.int32(par * 3 * CH)
            wbuf[pl.ds(wb + np.int32(0 * CH) + p16, 16)] = wx
            wbuf[pl.ds(wb + np.int32(1 * CH) + p16, 16)] = wy
            wbuf[pl.ds(wb + np.int32(2 * CH) + p16, 16)] = wz

            t2c = cy * P2
            t2f = fy * P2
            t3c = cz * P3
            t3f = fz * P3
            q_cc = t2c ^ t3c
            q_fc = t2f ^ t3c
            q_cf = t2c ^ t3f
            q_ff = t2f ^ t3f

            hs = (cx ^ q_cc, cx ^ q_fc, fx ^ q_fc, fx ^ q_cc,
                  cx ^ q_cf, cx ^ q_ff, fx ^ q_ff, fx ^ q_cf)
            for c in range(8):
                idxbuf[pl.ds(np.int32(par * 8 * CH + c * CH) + p16, 16)] = (hs[c] & MASK) + off_l
            return ()

        lax.fori_loop(0, NGRP, grp, (), unroll=False)

    def fire(l, par):
        b = par * 8 * CH
        return pltpu.async_copy(table_hbm.at[idxbuf.at[pl.ds(b, 8 * CH)]],
                                rows.at[pl.ds(b, 8 * CH), :], sems[par])

    def interp_level(l, par):
        two_l = np.int32(2 * l)

        def grp(h, _):
            p8 = h * np.int32(8)
            widx = p8 + dup
            wb = np.int32(par * 3 * CH)
            wx = plsc.load_gather(wbuf, [widx + (wb + np.int32(0 * CH))])
            wy = plsc.load_gather(wbuf, [widx + (wb + np.int32(1 * CH))])
            wz = plsc.load_gather(wbuf, [widx + (wb + np.int32(2 * CH))])
            rwx = 1.0 - wx
            rwy = 1.0 - wy
            rwz = 1.0 - wz

            f = []
            for c in range(8):
                rowflat = widx + np.int32(par * 8 * CH + c * CH)
                f.append(plsc.load_gather(rows, [rowflat, fpat]))

            f03 = f[0] * wx + f[3] * rwx
            f12 = f[1] * wx + f[2] * rwx
            f56 = f[5] * wx + f[6] * rwx
            f47 = f[4] * wx + f[7] * rwx
            f0312 = f03 * wy + f12 * rwy
            f4756 = f47 * wy + f56 * rwy
            enc = f0312 * wz + f4756 * rwz

            colidx = fpat + two_l
            plsc.store_scatter(outbuf, [widx, colidx], enc)
            return ()

        lax.fori_loop(0, 2 * NGRP, grp, (), unroll=False)

    def chunk(ci, _):
        base = wid * np.int32(NPT) + ci * np.int32(CH)
        pltpu.sync_copy(in_hbm.at[pl.ds(base, CH), :], coords)

        hash_level(0, 0)
        d = fire(0, 0)
        descs = [d]
        for l in range(1, NUM_LEVELS):
            par = l & 1
            hash_level(l, par)
            descs.append(fire(l, par))
            descs[l - 1].wait()
            interp_level(l - 1, (l - 1) & 1)
        descs[NUM_LEVELS - 1].wait()
        interp_level(NUM_LEVELS - 1, (NUM_LEVELS - 1) & 1)

        pltpu.sync_copy(outbuf, out_hbm.at[pl.ds(base, CH), :])
        return ()

    lax.fori_loop(0, NCHUNK, chunk, (), unroll=False)


@jax.jit
def kernel(in_tensor, hash_table):
    with jax.enable_x64(False):
        return _launch(in_tensor, hash_table)


def _launch(in_tensor, hash_table):
    mesh = plsc.VectorSubcoreMesh(core_axis_name="c", subcore_axis_name="s")
    f = pl.kernel(
        _body,
        out_type=jax.ShapeDtypeStruct((N_POINTS, NUM_LEVELS * FEATS),
                                      jnp.float32),
        mesh=mesh,
        scratch_types=[
            pltpu.VMEM((CH, 3), jnp.float32),
            pltpu.VMEM((2 * 8 * CH,), jnp.int32),
            pltpu.VMEM((2 * 8 * CH, FEATS), jnp.float32),
            pltpu.VMEM((2 * 3 * CH,), jnp.float32),
            pltpu.VMEM((CH, NUM_LEVELS * FEATS), jnp.float32),
            pltpu.SemaphoreType.DMA,
            pltpu.SemaphoreType.DMA,
        ],
        compiler_params=pltpu.CompilerParams(needs_layout_passes=False,
                                             use_tc_tiling_on_sc=False),
    )
    return f(in_tensor, hash_table)


# speedup vs baseline: 6.4463x; 5.1517x over previous
import jax
import jax.numpy as jnp
import numpy as np
from jax import lax
from jax.experimental import pallas as pl
from jax.experimental.pallas import tpu as pltpu
from jax.experimental.pallas import tpu_sc as plsc

NUM_LEVELS = 16
MIN_RES = 16
MAX_RES = 4096
LOG2_HASHMAP_SIZE = 19
FEATS = 2
TABLE_SIZE = 2 ** LOG2_HASHMAP_SIZE
N_POINTS = 262144
OUT_F = NUM_LEVELS * FEATS

NC = 2
NS = 16
NW = NC * NS
NPT = N_POINTS // NW
CH = 512
NCHUNK = NPT // CH
NGRP = CH // 16

MASK = np.int32(TABLE_SIZE - 1)
P2 = np.int32(np.uint32(2654435761))
P3 = np.int32(805459861)

_growth = np.exp((np.log(MAX_RES) - np.log(MIN_RES)) / (NUM_LEVELS - 1))
SCALES = [np.float32(np.floor(MIN_RES * _growth ** l)) for l in range(NUM_LEVELS)]


def _body(in_hbm, t0_hbm, t1_hbm, out_hbm,
          coords, idxbuf, rows0, rows1, wbuf, outbuf, sem0, sem1, sem2):
    sems = (sem0, sem1)
    wid = lax.axis_index("s") * NC + lax.axis_index("c")

    iota = jnp.arange(16, dtype=jnp.int32)
    col0 = jnp.zeros((16,), jnp.int32)
    col1 = jnp.full((16,), 1, jnp.int32)
    col2 = jnp.full((16,), 2, jnp.int32)

    def hash_level(l, par):
        scale = SCALES[l]
        off_l = np.int32(l * TABLE_SIZE)

        def grp(g, _):
            p16 = g * np.int32(16)
            rowidx = p16 + iota
            xv = plsc.load_gather(coords, [rowidx, col0])
            yv = plsc.load_gather(coords, [rowidx, col1])
            zv = plsc.load_gather(coords, [rowidx, col2])
            one = np.int32(1)
            zero = np.int32(0)

            sx = xv * scale
            fx = sx.astype(jnp.int32)
            wx = sx - fx.astype(jnp.float32)
            cx = fx + jnp.where(wx > 0, one, zero)

            sy = yv * scale
            fy = sy.astype(jnp.int32)
            wy = sy - fy.astype(jnp.float32)
            cy = fy + jnp.where(wy > 0, one, zero)

            sz = zv * scale
            fz = sz.astype(jnp.int32)
            wz = sz - fz.astype(jnp.float32)
            cz = fz + jnp.where(wz > 0, one, zero)

            wb = np.int32(par * 3 * CH)
            wbuf[pl.ds(wb + np.int32(0 * CH) + p16, 16)] = wx
            wbuf[pl.ds(wb + np.int32(1 * CH) + p16, 16)] = wy
            wbuf[pl.ds(wb + np.int32(2 * CH) + p16, 16)] = wz

            t2c = cy * P2
            t2f = fy * P2
            t3c = cz * P3
            t3f = fz * P3
            q_cc = t2c ^ t3c
            q_fc = t2f ^ t3c
            q_cf = t2c ^ t3f
            q_ff = t2f ^ t3f

            hs = (cx ^ q_cc, cx ^ q_fc, fx ^ q_fc, fx ^ q_cc,
                  cx ^ q_cf, cx ^ q_ff, fx ^ q_ff, fx ^ q_cf)
            ib = np.int32(par * 8 * CH)
            for c in range(8):
                idxbuf[pl.ds(ib + np.int32(c * CH) + p16, 16)] = \
                    (hs[c] & MASK) + off_l
            return ()

        lax.fori_loop(0, NGRP, grp, (), unroll=False)

    def fire(par):
        b = par * 8 * CH
        idx = idxbuf.at[pl.ds(b, 8 * CH)]
        dst = pl.ds(b, 8 * CH)
        return (pltpu.async_copy(t0_hbm.at[idx], rows0.at[dst], sems[par]),
                pltpu.async_copy(t1_hbm.at[idx], rows1.at[dst], sems[par]))

    def interp_level(l, par):
        two_l = np.int32(2 * l)

        def grp(g, _):
            p16 = g * np.int32(16)
            wb = np.int32(par * 3 * CH)
            wx = wbuf[pl.ds(wb + np.int32(0 * CH) + p16, 16)]
            wy = wbuf[pl.ds(wb + np.int32(1 * CH) + p16, 16)]
            wz = wbuf[pl.ds(wb + np.int32(2 * CH) + p16, 16)]
            rwx = 1.0 - wx
            rwy = 1.0 - wy
            rwz = 1.0 - wz

            rb = np.int32(par * 8 * CH)
            for j, rows in ((0, rows0), (1, rows1)):
                f = [rows[pl.ds(rb + np.int32(c * CH) + p16, 16)]
                     for c in range(8)]
                f03 = f[0] * wx + f[3] * rwx
                f12 = f[1] * wx + f[2] * rwx
                f56 = f[5] * wx + f[6] * rwx
                f47 = f[4] * wx + f[7] * rwx
                f0312 = f03 * wy + f12 * rwy
                f4756 = f47 * wy + f56 * rwy
                enc = f0312 * wz + f4756 * rwz
                outbuf[pl.ds(np.int32((2 * l + j) * CH) + p16, 16)] = enc
            return ()

        lax.fori_loop(0, NGRP, grp, (), unroll=False)

    def chunk(ci, _):
        base = wid * np.int32(NPT) + ci * np.int32(CH)
        pltpu.sync_copy(in_hbm.at[pl.ds(base, CH), :], coords)

        hash_level(0, 0)
        descs = [fire(0)]
        for l in range(1, NUM_LEVELS):
            par = l & 1
            hash_level(l, par)
            descs.append(fire(par))
            for d in descs[l - 1]:
                d.wait()
            interp_level(l - 1, (l - 1) & 1)
        for d in descs[NUM_LEVELS - 1]:
            d.wait()
        interp_level(NUM_LEVELS - 1, (NUM_LEVELS - 1) & 1)

        odescs = []
        for k in range(OUT_F):
            odescs.append(pltpu.async_copy(
                outbuf.at[pl.ds(k * CH, CH)],
                out_hbm.at[pl.ds(np.int32(k * N_POINTS) + base, CH)], sem2))
        for d in odescs:
            d.wait()
        return ()

    lax.fori_loop(0, NCHUNK, chunk, (), unroll=False)


@jax.jit
def kernel(in_tensor, hash_table):
    with jax.enable_x64(False):
        return _launch(in_tensor, hash_table)


def _launch(in_tensor, hash_table):
    mesh = plsc.VectorSubcoreMesh(core_axis_name="c", subcore_axis_name="s")
    f = pl.kernel(
        _body,
        out_type=jax.ShapeDtypeStruct((OUT_F * N_POINTS,), jnp.float32),
        mesh=mesh,
        scratch_types=[
            pltpu.VMEM((CH, 3), jnp.float32),
            pltpu.VMEM((2 * 8 * CH,), jnp.int32),
            pltpu.VMEM((2 * 8 * CH,), jnp.float32),
            pltpu.VMEM((2 * 8 * CH,), jnp.float32),
            pltpu.VMEM((2 * 3 * CH,), jnp.float32),
            pltpu.VMEM((OUT_F * CH,), jnp.float32),
            pltpu.SemaphoreType.DMA,
            pltpu.SemaphoreType.DMA,
            pltpu.SemaphoreType.DMA,
        ],
        compiler_params=pltpu.CompilerParams(needs_layout_passes=False,
                                             use_tc_tiling_on_sc=False),
    )
    t0 = hash_table[:, 0]
    t1 = hash_table[:, 1]
    out = f(in_tensor, t0, t1)
    return out.reshape(OUT_F, N_POINTS).T
